# baseline (speedup 1.0000x reference)
import functools

import jax
import jax.numpy as jnp
from jax import lax
from jax.experimental import pallas as pl
from jax.experimental.pallas import tpu as pltpu
from jax.experimental.pallas import tpu_sc as plsc

NC = 2
NS = 16
NW = NC * NS
NSLOT = 5


def _mm_kernel(x_ref, w_ref, o_ref):
    o_ref[...] = jnp.dot(x_ref[...], w_ref[...],
                         preferred_element_type=jnp.float32)


def _tc_matmul(x, w, bm):
    m, k = x.shape
    n = w.shape[1]
    grid = (m // bm,)
    return pl.pallas_call(
        _mm_kernel,
        grid=grid,
        in_specs=[
            pl.BlockSpec((bm, k), lambda i: (i, 0)),
            pl.BlockSpec((k, n), lambda i: (0, 0)),
        ],
        out_specs=pl.BlockSpec((bm, n), lambda i: (i, 0)),
        out_shape=jax.ShapeDtypeStruct((m, n), jnp.float32),
    )(x, w)


def _mm2_kernel(a_ref, b_ref, w_ref, o_ref):
    o_ref[...] = jnp.dot(a_ref[...] + b_ref[...], w_ref[...],
                         preferred_element_type=jnp.float32)


def _tc_add_matmul(a, b, w, bm):
    m, k = a.shape
    n = w.shape[1]
    grid = (m // bm,)
    return pl.pallas_call(
        _mm2_kernel,
        grid=grid,
        in_specs=[
            pl.BlockSpec((bm, k), lambda i: (i, 0)),
            pl.BlockSpec((bm, k), lambda i: (i, 0)),
            pl.BlockSpec((k, n), lambda i: (0, 0)),
        ],
        out_specs=pl.BlockSpec((bm, n), lambda i: (i, 0)),
        out_shape=jax.ShapeDtypeStruct((m, n), jnp.float32),
    )(a, b, w)


def _zzt_kernel(z_blk_ref, z_all_ref, o_ref):
    o_ref[...] = lax.dot_general(
        z_blk_ref[...], z_all_ref[...],
        dimension_numbers=(((1,), (1,)), ((), ())),
        preferred_element_type=jnp.float32)


def _tc_zzt(z, bm):
    nb, d = z.shape
    grid = (nb // bm,)
    return pl.pallas_call(
        _zzt_kernel,
        grid=grid,
        in_specs=[
            pl.BlockSpec((bm, d), lambda i: (i, 0)),
            pl.BlockSpec((nb, d), lambda i: (0, 0)),
        ],
        out_specs=pl.BlockSpec((bm, nb), lambda i: (i, 0)),
        out_shape=jax.ShapeDtypeStruct((nb, nb), jnp.float32),
    )(z, z)


def _sc_segment_sum(h, src3, dst3, zeros, n_pad, chunk):
    _, n_ch, _ = src3.shape
    d = h.shape[1]
    assert src3.shape == dst3.shape == (NW, n_ch, chunk)
    assert chunk % 8 == 0 and chunk <= 128
    assert n_pad % (NS * 8) == 0
    rows_per_tile = n_pad // NS

    mesh = plsc.VectorSubcoreMesh(core_axis_name="c", subcore_axis_name="s")

    @functools.partial(
        pl.kernel,
        out_type=jax.ShapeDtypeStruct((NC, n_pad, d), jnp.float32),
        mesh=mesh,
        scratch_types=[
            pltpu.VMEM((n_ch, chunk), jnp.int32),
            pltpu.VMEM((n_ch, chunk), jnp.int32),
            pltpu.VMEM((chunk, d), jnp.float32),
            pltpu.VMEM_SHARED((n_pad, d), jnp.float32),
            pltpu.SemaphoreType.DMA,
            pltpu.SemaphoreType.DMA,
        ],
    )
    def seg(h_hbm, src_hbm, dst_hbm, z_hbm, out_hbm, sidx, didx, rows, acc,
            sem, isem):
        c = lax.axis_index("c")
        s = lax.axis_index("s")
        wid = c * NS + s
        row0 = pl.multiple_of(s * rows_per_tile, 8)
        pltpu.async_copy(src_hbm.at[wid], sidx, isem)
        pltpu.async_copy(dst_hbm.at[wid], didx, isem)
        pltpu.sync_copy(z_hbm, acc.at[pl.ds(row0, rows_per_tile)])
        pltpu.make_async_copy(src_hbm.at[wid], sidx, isem).wait()
        pltpu.make_async_copy(dst_hbm.at[wid], didx, isem).wait()
        plsc.subcore_barrier()

        def body(j, carry):
            pltpu.async_copy(h_hbm.at[sidx.at[j]], rows, sem).wait()
            pltpu.sync_copy(rows, acc.at[didx.at[j]], add=True)
            return carry

        lax.fori_loop(0, n_ch, body, 0)
        plsc.subcore_barrier()
        pltpu.sync_copy(acc.at[pl.ds(row0, rows_per_tile)],
                        out_hbm.at[c].at[pl.ds(row0, rows_per_tile)])

    return seg(h, src3, dst3, zeros)


def _sc_gather_add_relu(q0, q1, nb):
    b = nb.shape[0]
    d = q0.shape[1]
    assert b % NW == 0
    b_per_w = b // NW
    assert b_per_w % 8 == 0 and b_per_w <= 128 and d % 16 == 0

    mesh = plsc.VectorSubcoreMesh(core_axis_name="c", subcore_axis_name="s")

    @functools.partial(
        pl.kernel,
        out_type=jax.ShapeDtypeStruct((b, d), jnp.float32),
        mesh=mesh,
        scratch_types=[
            pltpu.VMEM((b_per_w,), jnp.int32),
            pltpu.VMEM((b_per_w, d), jnp.float32),
            pltpu.VMEM((b_per_w, d), jnp.float32),
            pltpu.SemaphoreType.DMA,
        ],
    )
    def gat(q0_hbm, q1_hbm, nb_hbm, out_hbm, idx, b0, b1, sem):
        c = lax.axis_index("c")
        s = lax.axis_index("s")
        wid = c * NS + s
        base = pl.multiple_of(wid * b_per_w, 8)
        pltpu.sync_copy(nb_hbm.at[pl.ds(base, b_per_w)], idx)
        pltpu.async_copy(q0_hbm.at[idx], b0, sem).wait()
        pltpu.async_copy(q1_hbm.at[idx], b1, sem).wait()

        def body(i, carry):
            for j in range(d // 16):
                sl = pl.ds(j * 16, 16)
                v = b0[i, sl] + b1[i, sl]
                b0[i, sl] = jnp.maximum(v, 0.0)
            return carry

        lax.fori_loop(0, b_per_w, body, 0)
        pltpu.sync_copy(b0, out_hbm.at[pl.ds(base, b_per_w)])

    return gat(q0, q1, nb)


def kernel(adj, features, nodes_batch, W0, W1):
    n_nodes = features.shape[0]
    n_pad = ((n_nodes + NS * 8 - 1) // (NS * 8)) * (NS * 8)
    chunk = 128
    e = adj.shape[1]
    e_per_w = e // NW
    e_pad = ((e_per_w + chunk - 1) // chunk) * chunk
    n_ch = e_pad // chunk
    src2 = adj[0].astype(jnp.int32).reshape(NW, e_per_w)
    dst2 = adj[1].astype(jnp.int32).reshape(NW, e_per_w)
    pad_s = jnp.zeros((NW, e_pad - e_per_w), jnp.int32)
    pad_d = jnp.full((NW, e_pad - e_per_w), n_nodes, jnp.int32)
    src3 = jnp.concatenate([src2, pad_s], axis=1).reshape(NW, n_ch, chunk)
    dst3 = jnp.concatenate([dst2, pad_d], axis=1).reshape(NW, n_ch, chunk)
    nb = nodes_batch.astype(jnp.int32)

    hidden_dim = W0.shape[1]
    emb = W1.shape[1]
    w1p = jnp.pad(W1, ((0, 0), (0, hidden_dim - emb)))

    zeros_h = jnp.zeros((n_pad // NS, hidden_dim), jnp.float32)

    fpad = jnp.pad(features, ((0, n_pad - n_nodes), (0, 0)))
    h0 = _tc_matmul(fpad, W0, bm=n_pad // NS)
    p = _sc_segment_sum(h0, src3, dst3, zeros_h, n_pad, chunk)
    h1 = _tc_add_matmul(p[0], p[1], w1p, bm=n_pad // NS)
    q = _sc_segment_sum(h1, src3, dst3, zeros_h, n_pad, chunk)
    z = _sc_gather_add_relu(q[0], q[1], nb)
    return _tc_zzt(z, bm=256)


# speedup vs baseline: 1.7421x; 1.7421x over previous
import functools

import jax
import jax.numpy as jnp
from jax import lax
from jax.experimental import pallas as pl
from jax.experimental.pallas import tpu as pltpu
from jax.experimental.pallas import tpu_sc as plsc

NC = 2
NS = 16
NW = NC * NS
NSLOT = 5


def _mm_kernel(x_ref, w_ref, o_ref):
    o_ref[...] = jnp.dot(x_ref[...], w_ref[...],
                         preferred_element_type=jnp.float32)


def _tc_matmul(x, w, bm):
    m, k = x.shape
    n = w.shape[1]
    grid = (m // bm,)
    return pl.pallas_call(
        _mm_kernel,
        grid=grid,
        in_specs=[
            pl.BlockSpec((bm, k), lambda i: (i, 0)),
            pl.BlockSpec((k, n), lambda i: (0, 0)),
        ],
        out_specs=pl.BlockSpec((bm, n), lambda i: (i, 0)),
        out_shape=jax.ShapeDtypeStruct((m, n), jnp.float32),
    )(x, w)


def _mm2_kernel(a_ref, b_ref, w_ref, o_ref):
    o_ref[...] = jnp.dot(a_ref[...] + b_ref[...], w_ref[...],
                         preferred_element_type=jnp.float32)


def _tc_add_matmul(a, b, w, bm):
    m, k = a.shape
    n = w.shape[1]
    grid = (m // bm,)
    return pl.pallas_call(
        _mm2_kernel,
        grid=grid,
        in_specs=[
            pl.BlockSpec((bm, k), lambda i: (i, 0)),
            pl.BlockSpec((bm, k), lambda i: (i, 0)),
            pl.BlockSpec((k, n), lambda i: (0, 0)),
        ],
        out_specs=pl.BlockSpec((bm, n), lambda i: (i, 0)),
        out_shape=jax.ShapeDtypeStruct((m, n), jnp.float32),
    )(a, b, w)


def _zzt_kernel(z_blk_ref, z_all_ref, o_ref):
    o_ref[...] = lax.dot_general(
        z_blk_ref[...], z_all_ref[...],
        dimension_numbers=(((1,), (1,)), ((), ())),
        preferred_element_type=jnp.float32)


def _tc_zzt(z, bm):
    nb, d = z.shape
    grid = (nb // bm,)
    return pl.pallas_call(
        _zzt_kernel,
        grid=grid,
        in_specs=[
            pl.BlockSpec((bm, d), lambda i: (i, 0)),
            pl.BlockSpec((nb, d), lambda i: (0, 0)),
        ],
        out_specs=pl.BlockSpec((bm, nb), lambda i: (i, 0)),
        out_shape=jax.ShapeDtypeStruct((nb, nb), jnp.float32),
    )(z, z)


def _sc_segment_sum(h, src3, dst3, zeros, n_pad, chunk):
    _, n_ch, _ = src3.shape
    d = h.shape[1]
    assert src3.shape == dst3.shape == (NW, n_ch, chunk)
    assert chunk % 8 == 0 and chunk <= 128
    assert n_pad % (NS * 8) == 0
    rows_per_tile = n_pad // NS

    mesh = plsc.VectorSubcoreMesh(core_axis_name="c", subcore_axis_name="s")

    @functools.partial(
        pl.kernel,
        out_type=jax.ShapeDtypeStruct((NC, n_pad, d), jnp.float32),
        mesh=mesh,
        scratch_types=[
            pltpu.VMEM((n_ch, chunk), jnp.int32),
            pltpu.VMEM((n_ch, chunk), jnp.int32),
            pltpu.VMEM((chunk, d), jnp.float32),
            pltpu.VMEM_SHARED((n_pad, d), jnp.float32),
            pltpu.SemaphoreType.DMA,
            pltpu.SemaphoreType.DMA,
        ],
    )
    def seg(h_hbm, src_hbm, dst_hbm, z_hbm, out_hbm, sidx, didx, rows, acc,
            sem, isem):
        c = lax.axis_index("c")
        s = lax.axis_index("s")
        wid = c * NS + s
        row0 = pl.multiple_of(s * rows_per_tile, 8)
        pltpu.async_copy(src_hbm.at[wid], sidx, isem)
        pltpu.async_copy(dst_hbm.at[wid], didx, isem)
        pltpu.sync_copy(z_hbm, acc.at[pl.ds(row0, rows_per_tile)])
        pltpu.make_async_copy(src_hbm.at[wid], sidx, isem).wait()
        pltpu.make_async_copy(dst_hbm.at[wid], didx, isem).wait()
        plsc.subcore_barrier()

        def body(j, carry):
            pltpu.async_copy(h_hbm.at[sidx.at[j]], rows, sem).wait()
            pltpu.sync_copy(rows, acc.at[didx.at[j]], add=True)
            return carry

        lax.fori_loop(0, n_ch, body, 0)
        plsc.subcore_barrier()
        pltpu.sync_copy(acc.at[pl.ds(row0, rows_per_tile)],
                        out_hbm.at[c].at[pl.ds(row0, rows_per_tile)])

    return seg(h, src3, dst3, zeros)


def _sc_segment_sum_filtered(h, src3, dst3, nbatch, zeros, n_pad, chunk):
    _, n_ch, _ = src3.shape
    d = h.shape[1]
    nb_n = nbatch.shape[0]
    assert src3.shape == dst3.shape == (NW, n_ch, chunk)
    assert chunk % 16 == 0 and chunk <= 128 and nb_n % 16 == 0
    assert n_pad % 16 == 0
    nb_pad = nb_n + NS * 8
    assert nb_pad % (NS * 8) == 0
    rows_per_tile = nb_pad // NS
    e_pad = n_ch * chunk
    cpr = chunk // 16
    nb_per_w = nb_n // NW

    mesh = plsc.VectorSubcoreMesh(core_axis_name="c", subcore_axis_name="s")

    @functools.partial(
        pl.kernel,
        out_type=(jax.ShapeDtypeStruct((NC, nb_pad, d), jnp.float32),
                  jax.ShapeDtypeStruct((nb_n,), jnp.int32)),
        mesh=mesh,
        compiler_params=pltpu.CompilerParams(needs_layout_passes=False),
        scratch_types=[
            pltpu.VMEM((n_ch, chunk), jnp.int32),
            pltpu.VMEM((n_ch, chunk), jnp.int32),
            pltpu.VMEM((chunk, d), jnp.float32),
            pltpu.VMEM((n_pad,), jnp.int32),
            pltpu.VMEM((nb_n,), jnp.int32),
            pltpu.VMEM((nb_per_w,), jnp.int32),
            pltpu.VMEM((e_pad + chunk,), jnp.int32),
            pltpu.VMEM((e_pad + chunk,), jnp.int32),
            pltpu.VMEM((chunk,), jnp.int32),
            pltpu.VMEM_SHARED((nb_pad, d), jnp.float32),
            pltpu.SemaphoreType.DMA,
            pltpu.SemaphoreType.DMA,
        ],
    )
    def seg(h_hbm, src_hbm, dst_hbm, nb_hbm, z_hbm, out_hbm, slot_hbm, sidx,
            didx, rows, flag, nbbuf, slbuf, fsrc, fdst, cidx, acc, sem, isem):
        c = lax.axis_index("c")
        s = lax.axis_index("s")
        wid = c * NS + s
        row0 = pl.multiple_of(s * rows_per_tile, 8)
        pltpu.async_copy(src_hbm.at[wid], sidx, isem)
        pltpu.async_copy(dst_hbm.at[wid], didx, isem)
        pltpu.async_copy(nb_hbm, nbbuf, isem)
        pltpu.sync_copy(z_hbm, acc.at[pl.ds(row0, rows_per_tile)])

        zero16 = jnp.zeros((16,), jnp.int32)

        def zf(i, carry):
            flag[pl.ds(i * 16, 16)] = zero16
            return carry

        lax.fori_loop(0, n_pad // 16, zf, 0)
        pltpu.make_async_copy(src_hbm.at[wid], sidx, isem).wait()
        pltpu.make_async_copy(dst_hbm.at[wid], didx, isem).wait()
        pltpu.make_async_copy(nb_hbm, nbbuf, isem).wait()

        iota16 = lax.iota(jnp.int32, 16)

        def mark(i, carry):
            v = nbbuf[pl.ds(i * 16, 16)]
            plsc.store_scatter(flag, [v], i * 16 + iota16 + 1)
            return carry

        lax.fori_loop(0, nb_n // 16, mark, 0)

        nb0 = pl.multiple_of(wid * nb_per_w, 8)
        for k in range(nb_per_w // 16):
            v = nbbuf[pl.ds(nb0 + k * 16, 16)]
            slbuf[pl.ds(k * 16, 16)] = plsc.load_gather(flag, [v]) - 1
        pltpu.sync_copy(slbuf, slot_hbm.at[pl.ds(nb0, nb_per_w)])

        def compact(i, cnt):
            r = i // cpr
            k = i % cpr
            sv = sidx[r, pl.ds(k * 16, 16)]
            dv = didx[r, pl.ds(k * 16, 16)]
            fl = plsc.load_gather(flag, [dv])
            m = fl > 0
            mi = jnp.where(m, 1, 0).astype(jnp.int32)
            pos = cnt + plsc.cumsum(mi) - 1
            plsc.store_scatter(fsrc, [pos], sv, mask=m)
            plsc.store_scatter(fdst, [pos], fl - 1, mask=m)
            return cnt + jnp.sum(mi)

        cnt = lax.fori_loop(0, e_pad // 16, compact, jnp.int32(0))

        scrap16 = jnp.full((16,), nb_n, jnp.int32)
        zero16i = jnp.zeros((16,), jnp.int32)
        for k in range(cpr):
            plsc.store_scatter(fsrc, [cnt + k * 16 + iota16], zero16i)
            plsc.store_scatter(fdst, [cnt + k * 16 + iota16], scrap16)
        n_f_ch = (cnt + (chunk - 1)) // chunk

        plsc.subcore_barrier()

        def body(j, carry):
            off = j * chunk
            pltpu.async_copy(h_hbm.at[fsrc.at[pl.ds(off, chunk)]], rows,
                             sem).wait()
            for k in range(cpr):
                cidx[pl.ds(k * 16, 16)] = fdst[pl.ds(off + k * 16, 16)]
            pltpu.sync_copy(rows, acc.at[cidx], add=True)
            return carry

        lax.fori_loop(0, n_f_ch, body, 0)
        plsc.subcore_barrier()
        pltpu.sync_copy(acc.at[pl.ds(row0, rows_per_tile)],
                        out_hbm.at[c].at[pl.ds(row0, rows_per_tile)])

    return seg(h, src3, dst3, nbatch, zeros)


def _sc_gather_add_relu(q0, q1, nb):
    b = nb.shape[0]
    d = q0.shape[1]
    assert b % NW == 0
    b_per_w = b // NW
    assert b_per_w % 8 == 0 and b_per_w <= 128 and d % 16 == 0

    mesh = plsc.VectorSubcoreMesh(core_axis_name="c", subcore_axis_name="s")

    @functools.partial(
        pl.kernel,
        out_type=jax.ShapeDtypeStruct((b, d), jnp.float32),
        mesh=mesh,
        scratch_types=[
            pltpu.VMEM((b_per_w,), jnp.int32),
            pltpu.VMEM((b_per_w, d), jnp.float32),
            pltpu.VMEM((b_per_w, d), jnp.float32),
            pltpu.SemaphoreType.DMA,
        ],
    )
    def gat(q0_hbm, q1_hbm, nb_hbm, out_hbm, idx, b0, b1, sem):
        c = lax.axis_index("c")
        s = lax.axis_index("s")
        wid = c * NS + s
        base = pl.multiple_of(wid * b_per_w, 8)
        pltpu.sync_copy(nb_hbm.at[pl.ds(base, b_per_w)], idx)
        pltpu.async_copy(q0_hbm.at[idx], b0, sem).wait()
        pltpu.async_copy(q1_hbm.at[idx], b1, sem).wait()

        def body(i, carry):
            for j in range(d // 16):
                sl = pl.ds(j * 16, 16)
                v = b0[i, sl] + b1[i, sl]
                b0[i, sl] = jnp.maximum(v, 0.0)
            return carry

        lax.fori_loop(0, b_per_w, body, 0)
        pltpu.sync_copy(b0, out_hbm.at[pl.ds(base, b_per_w)])

    return gat(q0, q1, nb)


def kernel(adj, features, nodes_batch, W0, W1):
    n_nodes = features.shape[0]
    n_pad = ((n_nodes + NS * 8 - 1) // (NS * 8)) * (NS * 8)
    chunk = 80
    e = adj.shape[1]
    e_per_w = e // NW
    e_pad = ((e_per_w + chunk - 1) // chunk) * chunk
    n_ch = e_pad // chunk
    src2 = adj[0].astype(jnp.int32).reshape(NW, e_per_w)
    dst2 = adj[1].astype(jnp.int32).reshape(NW, e_per_w)
    pad_s = jnp.zeros((NW, e_pad - e_per_w), jnp.int32)
    pad_d = jnp.full((NW, e_pad - e_per_w), n_nodes, jnp.int32)
    src3 = jnp.concatenate([src2, pad_s], axis=1).reshape(NW, n_ch, chunk)
    dst3 = jnp.concatenate([dst2, pad_d], axis=1).reshape(NW, n_ch, chunk)
    nb = nodes_batch.astype(jnp.int32)

    hidden_dim = W0.shape[1]
    emb = W1.shape[1]
    w1p = jnp.pad(W1, ((0, 0), (0, hidden_dim - emb)))

    zeros_h = jnp.zeros((n_pad // NS, hidden_dim), jnp.float32)

    fpad = jnp.pad(features, ((0, n_pad - n_nodes), (0, 0)))
    h0 = _tc_matmul(fpad, W0, bm=n_pad // NS)
    p = _sc_segment_sum(h0, src3, dst3, zeros_h, n_pad, chunk)
    h1 = _tc_add_matmul(p[0], p[1], w1p, bm=n_pad // NS)
    nbn = nb.shape[0]
    zeros_z = jnp.zeros(((nbn + NS * 8) // NS, hidden_dim), jnp.float32)
    q, slots = _sc_segment_sum_filtered(h1, src3, dst3, nb, zeros_z, n_pad,
                                        chunk)
    z = _sc_gather_add_relu(q[0], q[1], slots)
    return _tc_zzt(z, bm=256)
